# initial kernel scaffold (unmeasured)
import jax
import jax.numpy as jnp
from jax import lax
from jax.experimental import pallas as pl
from jax.experimental.pallas import tpu as pltpu


def kernel(
    x,
):
    def body(*refs):
        pass

    out_shape = jax.ShapeDtypeStruct(..., jnp.float32)
    return pl.pallas_call(body, out_shape=out_shape)(...)


# baseline (device time: 589499 ns/iter reference)
import jax
import jax.numpy as jnp
from jax import lax
from jax.experimental import pallas as pl
from jax.experimental.pallas import tpu as pltpu

N_DEV = 4
M = 4096
N = 2048
M_CHUNK = M // N_DEV


def kernel(x):

    def body(x_hbm, out_ref, x_stage, local_sem, rs_send, rs_recv, ag_send, ag_recv):
        my = lax.axis_index("i")
        left = lax.rem(my - 1 + N_DEV, N_DEV)
        right = lax.rem(my + 1, N_DEV)

        barrier_sem = pltpu.get_barrier_semaphore()
        for nbr in (left, right):
            pl.semaphore_signal(
                barrier_sem, inc=1,
                device_id=(nbr,), device_id_type=pl.DeviceIdType.MESH,
            )
        pl.semaphore_wait(barrier_sem, 2)

        def row(c):
            return pl.ds(c * M_CHUNK, M_CHUNK)

        for s in range(N_DEV - 1):
            sc = lax.rem(my - s + N_DEV, N_DEV)
            rc = lax.rem(my - s - 1 + N_DEV, N_DEV)
            if s == 0:
                src = x_hbm.at[0, row(sc), :]
            else:
                src = out_ref.at[row(sc), :]
            rdma = pltpu.make_async_remote_copy(
                src_ref=src,
                dst_ref=out_ref.at[row(sc), :],
                send_sem=rs_send.at[s],
                recv_sem=rs_recv.at[s],
                device_id=(right,),
                device_id_type=pl.DeviceIdType.MESH,
            )
            rdma.start()
            fetch = pltpu.make_async_copy(
                x_hbm.at[0, row(rc), :], x_stage, local_sem
            )
            fetch.start()
            rdma.wait()
            fetch.wait()
            out_ref[row(rc), :] = out_ref[row(rc), :] + x_stage[:, :]

        for s in range(N_DEV - 1):
            ac = lax.rem(my + 1 - s + N_DEV, N_DEV)
            rdma = pltpu.make_async_remote_copy(
                src_ref=out_ref.at[row(ac), :],
                dst_ref=out_ref.at[row(ac), :],
                send_sem=ag_send.at[s],
                recv_sem=ag_recv.at[s],
                device_id=(right,),
                device_id_type=pl.DeviceIdType.MESH,
            )
            rdma.start()
            rdma.wait()

    return pl.pallas_call(
        body,
        out_shape=jax.ShapeDtypeStruct((M, N), jnp.float32),
        in_specs=[pl.BlockSpec(memory_space=pl.ANY)],
        out_specs=pl.BlockSpec(memory_space=pltpu.VMEM),
        scratch_shapes=[
            pltpu.VMEM((M_CHUNK, N), jnp.float32),
            pltpu.SemaphoreType.DMA,
            pltpu.SemaphoreType.DMA((N_DEV - 1,)),
            pltpu.SemaphoreType.DMA((N_DEV - 1,)),
            pltpu.SemaphoreType.DMA((N_DEV - 1,)),
            pltpu.SemaphoreType.DMA((N_DEV - 1,)),
        ],
        compiler_params=pltpu.CompilerParams(
            collective_id=0,
            vmem_limit_bytes=56 * 1024 * 1024,
        ),
    )(x)


# device time: 319823 ns/iter; 1.8432x vs baseline; 1.8432x over previous
import jax
import jax.numpy as jnp
from jax import lax
from jax.experimental import pallas as pl
from jax.experimental.pallas import tpu as pltpu

N_DEV = 4
M = 4096
N = 2048
M_CHUNK = M // N_DEV
N_HALF = N // 2


def kernel(x):

    def body(x_hbm, out_ref, stage_cw, stage_ccw, fetch_sems,
             rs_send, rs_recv, ag_send, ag_recv):
        my = lax.axis_index("i")
        left = lax.rem(my - 1 + N_DEV, N_DEV)
        right = lax.rem(my + 1, N_DEV)

        barrier_sem = pltpu.get_barrier_semaphore()
        for nbr in (left, right):
            pl.semaphore_signal(
                barrier_sem, inc=1,
                device_id=(nbr,), device_id_type=pl.DeviceIdType.MESH,
            )
        pl.semaphore_wait(barrier_sem, 2)

        def row(c):
            return pl.ds(c * M_CHUNK, M_CHUNK)

        cw_cols = pl.ds(0, N_HALF)
        ccw_cols = pl.ds(N_HALF, N_HALF)

        for s in range(N_DEV - 1):
            sc_cw = lax.rem(my - s + N_DEV, N_DEV)
            rc_cw = lax.rem(my - s - 1 + N_DEV, N_DEV)
            sc_ccw = lax.rem(my + s, N_DEV)
            rc_ccw = lax.rem(my + s + 1, N_DEV)

            src_cw_ref = x_hbm.at[0] if s == 0 else out_ref
            rdma_cw = pltpu.make_async_remote_copy(
                src_ref=src_cw_ref.at[row(sc_cw), cw_cols],
                dst_ref=out_ref.at[row(sc_cw), cw_cols],
                send_sem=rs_send.at[0, s],
                recv_sem=rs_recv.at[0, s],
                device_id=(right,),
                device_id_type=pl.DeviceIdType.MESH,
            )
            src_ccw_ref = x_hbm.at[0] if s == 0 else out_ref
            rdma_ccw = pltpu.make_async_remote_copy(
                src_ref=src_ccw_ref.at[row(sc_ccw), ccw_cols],
                dst_ref=out_ref.at[row(sc_ccw), ccw_cols],
                send_sem=rs_send.at[1, s],
                recv_sem=rs_recv.at[1, s],
                device_id=(left,),
                device_id_type=pl.DeviceIdType.MESH,
            )
            rdma_cw.start()
            rdma_ccw.start()

            fetch_cw = pltpu.make_async_copy(
                x_hbm.at[0, row(rc_cw), cw_cols], stage_cw, fetch_sems.at[0]
            )
            fetch_ccw = pltpu.make_async_copy(
                x_hbm.at[0, row(rc_ccw), ccw_cols], stage_ccw, fetch_sems.at[1]
            )
            fetch_cw.start()
            fetch_ccw.start()

            rdma_cw.wait()
            fetch_cw.wait()
            out_ref[row(rc_cw), cw_cols] = (
                out_ref[row(rc_cw), cw_cols] + stage_cw[:, :]
            )
            rdma_ccw.wait()
            fetch_ccw.wait()
            out_ref[row(rc_ccw), ccw_cols] = (
                out_ref[row(rc_ccw), ccw_cols] + stage_ccw[:, :]
            )

        for s in range(N_DEV - 1):
            ac_cw = lax.rem(my + 1 - s + N_DEV, N_DEV)
            ac_ccw = lax.rem(my - 1 + s + N_DEV, N_DEV)
            rdma_cw = pltpu.make_async_remote_copy(
                src_ref=out_ref.at[row(ac_cw), cw_cols],
                dst_ref=out_ref.at[row(ac_cw), cw_cols],
                send_sem=ag_send.at[0, s],
                recv_sem=ag_recv.at[0, s],
                device_id=(right,),
                device_id_type=pl.DeviceIdType.MESH,
            )
            rdma_ccw = pltpu.make_async_remote_copy(
                src_ref=out_ref.at[row(ac_ccw), ccw_cols],
                dst_ref=out_ref.at[row(ac_ccw), ccw_cols],
                send_sem=ag_send.at[1, s],
                recv_sem=ag_recv.at[1, s],
                device_id=(left,),
                device_id_type=pl.DeviceIdType.MESH,
            )
            rdma_cw.start()
            rdma_ccw.start()
            rdma_cw.wait()
            rdma_ccw.wait()

    return pl.pallas_call(
        body,
        out_shape=jax.ShapeDtypeStruct((M, N), jnp.float32),
        in_specs=[pl.BlockSpec(memory_space=pl.ANY)],
        out_specs=pl.BlockSpec(memory_space=pltpu.VMEM),
        scratch_shapes=[
            pltpu.VMEM((M_CHUNK, N_HALF), jnp.float32),
            pltpu.VMEM((M_CHUNK, N_HALF), jnp.float32),
            pltpu.SemaphoreType.DMA((2,)),
            pltpu.SemaphoreType.DMA((2, N_DEV - 1)),
            pltpu.SemaphoreType.DMA((2, N_DEV - 1)),
            pltpu.SemaphoreType.DMA((2, N_DEV - 1)),
            pltpu.SemaphoreType.DMA((2, N_DEV - 1)),
        ],
        compiler_params=pltpu.CompilerParams(
            collective_id=0,
            vmem_limit_bytes=56 * 1024 * 1024,
        ),
    )(x)


# device time: 309428 ns/iter; 1.9051x vs baseline; 1.0336x over previous
import jax
import jax.numpy as jnp
from jax import lax
from jax.experimental import pallas as pl
from jax.experimental.pallas import tpu as pltpu

N_DEV = 4
M = 4096
N = 2048
M_CHUNK = M // N_DEV
N_HALF = N // 2
K = 2
SUB = M_CHUNK // K
N_STEP = N_DEV - 1
DIRS = (0, 1)


def kernel(x):

    def body(x_hbm, out_ref, stage_cw, stage_ccw, fetch_sems,
             rs_send, rs_recv, ag_send, ag_recv):
        my = lax.axis_index("i")
        left = lax.rem(my - 1 + N_DEV, N_DEV)
        right = lax.rem(my + 1, N_DEV)

        barrier_sem = pltpu.get_barrier_semaphore()
        for nbr in (left, right):
            pl.semaphore_signal(
                barrier_sem, inc=1,
                device_id=(nbr,), device_id_type=pl.DeviceIdType.MESH,
            )
        pl.semaphore_wait(barrier_sem, 2)

        def cols(d):
            return pl.ds(0, N_HALF) if d == 0 else pl.ds(N_HALF, N_HALF)

        def peer(d):
            return right if d == 0 else left

        def sc(d, s):
            off = -s if d == 0 else s
            return lax.rem(my + off + N_DEV, N_DEV)

        def rc(d, s):
            off = -s - 1 if d == 0 else s + 1
            return lax.rem(my + off + N_DEV, N_DEV)

        def ac(d, s):
            off = 1 - s if d == 0 else s - 1
            return lax.rem(my + off + N_DEV, N_DEV)

        def subrow(c, j):
            return pl.ds(c * M_CHUNK + j * SUB, SUB)

        def rs_desc(d, s, j):
            base = x_hbm.at[0] if s == 0 else out_ref
            c = sc(d, s)
            return pltpu.make_async_remote_copy(
                src_ref=base.at[subrow(c, j), cols(d)],
                dst_ref=out_ref.at[subrow(c, j), cols(d)],
                send_sem=rs_send.at[d, s, j],
                recv_sem=rs_recv.at[d, s, j],
                device_id=(peer(d),),
                device_id_type=pl.DeviceIdType.MESH,
            )

        def ag_desc(d, s, j):
            c = ac(d, s)
            return pltpu.make_async_remote_copy(
                src_ref=out_ref.at[subrow(c, j), cols(d)],
                dst_ref=out_ref.at[subrow(c, j), cols(d)],
                send_sem=ag_send.at[d, s, j],
                recv_sem=ag_recv.at[d, s, j],
                device_id=(peer(d),),
                device_id_type=pl.DeviceIdType.MESH,
            )

        stages = (stage_cw, stage_ccw)

        def start_fetch(s):
            fs = []
            for d in DIRS:
                f = pltpu.make_async_copy(
                    x_hbm.at[0, pl.ds(rc(d, s) * M_CHUNK, M_CHUNK), cols(d)],
                    stages[d],
                    fetch_sems.at[d],
                )
                f.start()
                fs.append(f)
            return fs

        rs_inflight = {}
        for d in DIRS:
            for j in range(K):
                r = rs_desc(d, 0, j)
                r.start()
                rs_inflight[(d, 0, j)] = r
        fetches = start_fetch(0)

        ag_inflight = {}
        for s in range(N_STEP):
            for j in range(K):
                for d in DIRS:
                    rs_inflight[(d, s, j)].wait()
                    if j == 0:
                        fetches[d].wait()
                    c = rc(d, s)
                    out_ref[subrow(c, j), cols(d)] = (
                        out_ref[subrow(c, j), cols(d)]
                        + stages[d][pl.ds(j * SUB, SUB), :]
                    )
                    if s < N_STEP - 1:
                        r = rs_desc(d, s + 1, j)
                        r.start()
                        rs_inflight[(d, s + 1, j)] = r
                    else:
                        a = ag_desc(d, 0, j)
                        a.start()
                        ag_inflight[(d, 0, j)] = a
            if s < N_STEP - 1:
                fetches = start_fetch(s + 1)

        for s in range(N_STEP):
            for j in range(K):
                for d in DIRS:
                    ag_inflight[(d, s, j)].wait()
                    if s < N_STEP - 1:
                        a = ag_desc(d, s + 1, j)
                        a.start()
                        ag_inflight[(d, s + 1, j)] = a

    return pl.pallas_call(
        body,
        out_shape=jax.ShapeDtypeStruct((M, N), jnp.float32),
        in_specs=[pl.BlockSpec(memory_space=pl.ANY)],
        out_specs=pl.BlockSpec(memory_space=pltpu.VMEM),
        scratch_shapes=[
            pltpu.VMEM((M_CHUNK, N_HALF), jnp.float32),
            pltpu.VMEM((M_CHUNK, N_HALF), jnp.float32),
            pltpu.SemaphoreType.DMA((2,)),
            pltpu.SemaphoreType.DMA((2, N_STEP, K)),
            pltpu.SemaphoreType.DMA((2, N_STEP, K)),
            pltpu.SemaphoreType.DMA((2, N_STEP, K)),
            pltpu.SemaphoreType.DMA((2, N_STEP, K)),
        ],
        compiler_params=pltpu.CompilerParams(
            collective_id=0,
            vmem_limit_bytes=56 * 1024 * 1024,
        ),
    )(x)
